# baseline (speedup 1.0000x reference)
import functools

import jax
import jax.numpy as jnp
from jax import lax
from jax.experimental import pallas as pl
from jax.experimental.pallas import tpu as pltpu
from jax.experimental.pallas import tpu_sc as plsc

_MAX_LEN = 512
_LANES = 16


def _positional_encodings(max_len, embed_dim):
    pos = jnp.arange(0, max_len, dtype=jnp.float32).reshape(-1, 1)
    skip = jnp.arange(0, embed_dim, 2, dtype=jnp.float32)
    denom = 10000.0 ** (skip / embed_dim)
    enc = jnp.zeros((max_len, embed_dim), dtype=jnp.float32)
    enc = enc.at[:, 0::2].set(jnp.sin(pos / denom))
    enc = enc.at[:, 1::2].set(jnp.cos(pos / denom))
    return enc


def kernel(input_ids, src_table):
    B, L = input_ids.shape
    V, D = src_table.shape
    ids_flat = input_ids.astype(jnp.int32).reshape(-1)
    enc_flat = _positional_encodings(_MAX_LEN, D)[:L].astype(
        jnp.float32).reshape(-1)

    info = plsc.get_sparse_core_info()
    NC, NS = info.num_cores, info.num_subcores
    NW = NC * NS
    assert B % (4 * NW) == 0, (B, NW)
    rows_per_w = B // NW
    n_pairs = rows_per_w // 2
    toks_w = rows_per_w * L
    assert D % _LANES == 0 and (L * D) % 128 == 0
    LD = L * D
    splits = [(o, min(128, L - o)) for o in range(0, L, 128)]

    mesh = plsc.VectorSubcoreMesh(core_axis_name="c", subcore_axis_name="s")

    @functools.partial(
        pl.kernel,
        mesh=mesh,
        compiler_params=pltpu.CompilerParams(use_tc_tiling_on_sc=False),
        out_type=jax.ShapeDtypeStruct((B * L * D // 128, 128), jnp.float32),
        scratch_types=[
            pltpu.VMEM((toks_w,), jnp.int32),
            pltpu.VMEM((2 * L, D), jnp.float32),
            pltpu.VMEM((2, 2 * LD // 128, 128), jnp.float32),
            pltpu.VMEM((LD,), jnp.float32),
            [pltpu.SemaphoreType.DMA] * 2,
            [pltpu.SemaphoreType.DMA] * 2,
        ],
    )
    def run(ids_hbm, enc_hbm, table_hbm, out_hbm, idx_all, rows_v, cbuf,
            enc_v, gsems, osems):
        wid = lax.axis_index("s") * NC + lax.axis_index("c")
        tok0 = wid * toks_w

        pltpu.sync_copy(enc_hbm, enc_v)
        pltpu.sync_copy(ids_hbm.at[pl.ds(tok0, toks_w)], idx_all)

        def start_gather(g, q):
            for (o, n) in splits:
                pltpu.async_copy(
                    table_hbm.at[idx_all.at[pl.ds(g * L + o, n)]],
                    rows_v.at[pl.ds(q * L + o, n)],
                    gsems[q],
                )

        def wait_gather(q):
            pltpu.make_async_copy(
                table_hbm.at[pl.ds(0, L)],
                rows_v.at[pl.ds(q * L, L)], gsems[q]).wait()

        def wait_out(cb):
            pltpu.make_async_copy(
                cbuf.at[cb], out_hbm.at[pl.ds(0, 2 * LD // 128)],
                osems[cb]).wait()

        start_gather(0, 0)
        start_gather(1, 1)

        def outer(t, carry):
            for cb in range(2):
                p = 2 * t + cb

                @pl.when(t >= 1)
                def _():
                    wait_out(cb)

                for q in range(2):
                    g = 2 * p + q
                    wait_gather(q)

                    def cbody(i, carry2, q=q, cb=cb):
                        ro = q * L + 2 * i
                        co = q * (LD // 128) + i
                        for h in range(2):
                            for k in range(D // _LANES):
                                c0 = 64 * h + k * _LANES
                                cbuf[cb, co, pl.ds(c0, _LANES)] = (
                                    rows_v[ro + h, pl.ds(k * _LANES, _LANES)]
                                    + enc_v[pl.ds(i * 128 + c0, _LANES)])
                        return carry2

                    lax.fori_loop(0, L // 2, cbody, 0)

                    @pl.when(g + 2 < rows_per_w)
                    def _():
                        start_gather(g + 2, q)

                pltpu.async_copy(
                    cbuf.at[cb],
                    out_hbm.at[pl.ds((tok0 + p * L) * D // 128,
                                     2 * LD // 128)],
                    osems[cb])
            return carry

        lax.fori_loop(0, n_pairs // 2, outer, 0)

        for cb in range(2):
            wait_out(cb)

    out2 = run(ids_flat, enc_flat, src_table)
    return out2.reshape(B, L, D)
